# baseline (speedup 1.0000x reference)
import jax
import jax.numpy as jnp
from jax import lax
from jax.experimental import pallas as pl
from jax.experimental.pallas import tpu as pltpu
from jax.experimental.pallas import tpu_sc as plsc

VOCAB = 1000000
EMBED_DIM = 64
MAX_SEQ_LEN = 2048
BATCH = 16
SEQ_LEN = 2048

NC = 2
NS = 16
NW = NC * NS
S_PER_W = SEQ_LEN // NW
ROWS_PER_W = BATCH * S_PER_W
LANES = 16


def _body(idx_hbm, table_hbm, pos_hbm, out_hbm, idx_v, rows_v, pos_v, sem):
    wid = lax.axis_index("s") * NC + lax.axis_index("c")
    s0 = wid * S_PER_W

    for b in range(BATCH):
        pltpu.sync_copy(idx_hbm.at[pl.ds(b * SEQ_LEN + s0, S_PER_W)],
                        idx_v.at[b // 2, pl.ds((b % 2) * S_PER_W, S_PER_W)])

    pltpu.sync_copy(pos_hbm.at[pl.ds(s0, S_PER_W)], pos_v)

    copies = []
    for j in range(ROWS_PER_W // 128):
        copies.append(pltpu.async_copy(
            table_hbm.at[idx_v.at[j]],
            rows_v.at[pl.ds(j * 128, 128)],
            sem))
    for c in copies:
        c.wait()

    def add_row(i, _):
        for j in range(EMBED_DIM // LANES):
            p = pos_v[i, pl.ds(j * LANES, LANES)]
            for b in range(BATCH):
                r = b * S_PER_W + i
                rows_v[r, pl.ds(j * LANES, LANES)] = (
                    rows_v[r, pl.ds(j * LANES, LANES)] + p)
        return 0

    lax.fori_loop(0, S_PER_W, add_row, 0)

    for b in range(BATCH):
        pltpu.sync_copy(rows_v.at[pl.ds(b * S_PER_W, S_PER_W)],
                        out_hbm.at[pl.ds(b * SEQ_LEN + s0, S_PER_W)])


@jax.jit
def _embed(flat_idx, token_table, pos_table):
    mesh = plsc.VectorSubcoreMesh(core_axis_name="c", subcore_axis_name="s")
    run = pl.kernel(
        _body,
        out_type=jax.ShapeDtypeStruct((BATCH * SEQ_LEN, EMBED_DIM),
                                      jnp.float32),
        mesh=mesh,
        scratch_types=[
            pltpu.VMEM((ROWS_PER_W // 128, 128), jnp.int32),
            pltpu.VMEM((ROWS_PER_W, EMBED_DIM), jnp.float32),
            pltpu.VMEM((S_PER_W, EMBED_DIM), jnp.float32),
            pltpu.SemaphoreType.DMA,
        ],
        compiler_params=pltpu.CompilerParams(use_tc_tiling_on_sc=False),
    )
    return run(flat_idx, token_table, pos_table)


def kernel(token_indices, token_table, pos_table):
    flat_idx = token_indices.astype(jnp.int32).reshape(-1)
    out = _embed(flat_idx, token_table, pos_table)
    return out.reshape(BATCH, SEQ_LEN, EMBED_DIM)


# speedup vs baseline: 1.5287x; 1.5287x over previous
import jax
import jax.numpy as jnp
from jax import lax
from jax.experimental import pallas as pl
from jax.experimental.pallas import tpu as pltpu
from jax.experimental.pallas import tpu_sc as plsc

VOCAB = 1000000
EMBED_DIM = 64
BATCH = 16
SEQ_LEN = 2048

NC = 2
NS = 16
NW = NC * NS
S_PER_W = SEQ_LEN // NW
HALF_B = BATCH // 2
HALF_ROWS = HALF_B * S_PER_W
LANES = 16


def _body(idx_hbm, table_hbm, pos_hbm, out_hbm, idx_v, rows_v, pos_v, sem):
    wid = lax.axis_index("s") * NC + lax.axis_index("c")
    s0 = wid * S_PER_W

    pltpu.sync_copy(pos_hbm.at[pl.ds(s0, S_PER_W)], pos_v)

    for h in range(2):
        for b in range(HALF_B):
            pltpu.sync_copy(
                idx_hbm.at[h * HALF_B + b, pl.ds(s0, S_PER_W)],
                idx_v.at[pl.ds(b * S_PER_W, S_PER_W)])

        def gather_group(g, _):
            vec = idx_v[pl.ds(g * LANES, LANES)]
            copies = []
            for j in range(LANES):
                v = vec[j]
                copies.append(pltpu.async_copy(
                    table_hbm.at[pl.ds(v, 1)],
                    rows_v.at[pl.ds(g * LANES + j, 1)],
                    sem))
            for c in copies:
                c.wait()
            return 0

        lax.fori_loop(0, HALF_ROWS // LANES, gather_group, 0)

        def add_row(i, _):
            for j in range(EMBED_DIM // LANES):
                p = pos_v[i, pl.ds(j * LANES, LANES)]
                for b in range(HALF_B):
                    r = b * S_PER_W + i
                    rows_v[r, pl.ds(j * LANES, LANES)] = (
                        rows_v[r, pl.ds(j * LANES, LANES)] + p)
            return 0

        lax.fori_loop(0, S_PER_W, add_row, 0)

        for b in range(HALF_B):
            pltpu.sync_copy(
                rows_v.at[pl.ds(b * S_PER_W, S_PER_W)],
                out_hbm.at[h * HALF_B + b, pl.ds(s0, S_PER_W)])


@jax.jit
def _embed(token_indices, token_table, pos_table):
    mesh = plsc.VectorSubcoreMesh(core_axis_name="c", subcore_axis_name="s")
    run = pl.kernel(
        _body,
        out_type=jax.ShapeDtypeStruct((BATCH, SEQ_LEN, EMBED_DIM),
                                      jnp.float32),
        mesh=mesh,
        scratch_types=[
            pltpu.VMEM((HALF_ROWS,), jnp.int32),
            pltpu.VMEM((HALF_ROWS, EMBED_DIM), jnp.float32),
            pltpu.VMEM((S_PER_W, EMBED_DIM), jnp.float32),
            pltpu.SemaphoreType.DMA,
        ],
        compiler_params=pltpu.CompilerParams(use_tc_tiling_on_sc=True),
    )
    return run(token_indices, token_table, pos_table)


def kernel(token_indices, token_table, pos_table):
    return _embed(token_indices.astype(jnp.int32), token_table, pos_table)


# speedup vs baseline: 1.6581x; 1.0847x over previous
import jax
import jax.numpy as jnp
from jax import lax
from jax.experimental import pallas as pl
from jax.experimental.pallas import tpu as pltpu
from jax.experimental.pallas import tpu_sc as plsc

VOCAB = 1000000
EMBED_DIM = 64
BATCH = 16
SEQ_LEN = 2048

NC = 2
NS = 16
NW = NC * NS
S_PER_W = SEQ_LEN // NW
HALF_B = BATCH // 2
HALF_ROWS = HALF_B * S_PER_W
LANES = 16


def _body(idx_hbm, table_hbm, pos_hbm, out_hbm, idx_v, rows_v, pos_v, sem):
    wid = lax.axis_index("s") * NC + lax.axis_index("c")
    s0 = wid * S_PER_W

    pltpu.sync_copy(pos_hbm.at[pl.ds(s0, S_PER_W)], pos_v)

    for h in range(2):
        for b in range(HALF_B):
            pltpu.sync_copy(
                idx_hbm.at[h * HALF_B + b, pl.ds(s0, S_PER_W)],
                idx_v.at[pl.ds(b * S_PER_W, S_PER_W)])

        NGROUP = HALF_ROWS // LANES
        WINDOW = 8

        def fire_group(g):
            vec = idx_v[pl.ds(g * LANES, LANES)]
            for j in range(LANES):
                v = vec[j]
                pltpu.async_copy(
                    table_hbm.at[pl.ds(v, 1)],
                    rows_v.at[pl.ds(g * LANES + j, 1)],
                    sem)

        def drain_rows(n):
            pltpu.make_async_copy(
                table_hbm.at[pl.ds(0, n)], rows_v.at[pl.ds(0, n)], sem
            ).wait()

        for g in range(WINDOW):
            fire_group(g)

        def gather_step(g, _):
            fire_group(g)
            drain_rows(LANES)
            return 0

        lax.fori_loop(WINDOW, NGROUP, gather_step, 0)
        drain_rows(WINDOW * LANES)

        def add_row(i, _):
            for j in range(EMBED_DIM // LANES):
                p = pos_v[i, pl.ds(j * LANES, LANES)]
                for b in range(HALF_B):
                    r = b * S_PER_W + i
                    rows_v[r, pl.ds(j * LANES, LANES)] = (
                        rows_v[r, pl.ds(j * LANES, LANES)] + p)
            return 0

        lax.fori_loop(0, S_PER_W, add_row, 0)

        for b in range(HALF_B):
            pltpu.sync_copy(
                rows_v.at[pl.ds(b * S_PER_W, S_PER_W)],
                out_hbm.at[h * HALF_B + b, pl.ds(s0, S_PER_W)])


@jax.jit
def _embed(token_indices, token_table, pos_table):
    mesh = plsc.VectorSubcoreMesh(core_axis_name="c", subcore_axis_name="s")
    run = pl.kernel(
        _body,
        out_type=jax.ShapeDtypeStruct((BATCH, SEQ_LEN, EMBED_DIM),
                                      jnp.float32),
        mesh=mesh,
        scratch_types=[
            pltpu.VMEM((HALF_ROWS,), jnp.int32),
            pltpu.VMEM((HALF_ROWS, EMBED_DIM), jnp.float32),
            pltpu.VMEM((S_PER_W, EMBED_DIM), jnp.float32),
            pltpu.SemaphoreType.DMA,
        ],
        compiler_params=pltpu.CompilerParams(use_tc_tiling_on_sc=True),
    )
    return run(token_indices, token_table, pos_table)


def kernel(token_indices, token_table, pos_table):
    return _embed(token_indices.astype(jnp.int32), token_table, pos_table)
